# baseline (speedup 1.0000x reference)
import functools

import jax
import jax.numpy as jnp
from jax import lax
from jax.experimental import pallas as pl
from jax.experimental.pallas import tpu as pltpu
from jax.experimental.pallas import tpu_sc as plsc

NQ = 4096
ND = 256
NK = 65536
K = 5

TK = 512
NEG = float(jnp.finfo(jnp.float32).min)
BIGF = 1e9


def _row_normalize(x):
    n = jnp.linalg.norm(x, axis=1, keepdims=True)
    return x / jnp.maximum(n, 1e-12)


def _topk_body(k_ref, q_ref, w_ref, i_ref, runv_ref, runi_ref):
    kk = pl.program_id(0)
    nk = pl.num_programs(0)

    @pl.when(kk == 0)
    def _init():
        runv_ref[...] = jnp.full((8, NQ), NEG, jnp.float32)
        runi_ref[...] = jnp.zeros((8, NQ), jnp.float32)

    simq = lax.dot_general(
        q_ref[...], k_ref[...], (((1,), (1,)), ((), ())),
        preferred_element_type=jnp.float32,
    )
    sim = jnp.transpose(simq, (1, 0))

    h = TK // 2
    a = sim[0:h, :]
    b = sim[h:TK, :]
    io = lax.broadcasted_iota(jnp.int32, (h, NQ), 0).astype(jnp.float32)
    ioh = io + float(h)
    ge = a >= b
    wv = jnp.where(ge, a, b)
    lv = jnp.where(ge, b, a)
    wi = jnp.where(ge, io, ioh)
    li = jnp.where(ge, ioh, io)
    base = (kk * TK).astype(jnp.float32)
    bvals = []
    bidxs = []
    for _ in range(K):
        m = jnp.max(wv, axis=0, keepdims=True)
        cand = jnp.where(wv == m, wi, BIGF)
        pos = jnp.min(cand, axis=0, keepdims=True)
        bvals.append(m)
        bidxs.append(pos + base)
        hit = wi == pos
        wv = jnp.where(hit, lv, wv)
        wi = jnp.where(hit, li, wi)
        lv = jnp.where(hit, NEG, lv)

    pad3neg = jnp.full((3, NQ), NEG, jnp.float32)
    pad3f = jnp.zeros((3, NQ), jnp.float32)
    cv = jnp.concatenate([runv_ref[...]] + bvals + [pad3neg], axis=0)
    ci = jnp.concatenate([runi_ref[...]] + bidxs + [pad3f], axis=0)
    iota16 = lax.broadcasted_iota(jnp.int32, (16, NQ), 0).astype(jnp.float32)
    nv = []
    ni = []
    for _ in range(K):
        m = jnp.max(cv, axis=0, keepdims=True)
        pos = jnp.min(jnp.where(cv == m, iota16, BIGF), axis=0, keepdims=True)
        hit = iota16 == pos
        nv.append(m)
        ni.append(jnp.sum(jnp.where(hit, ci, 0.0), axis=0, keepdims=True))
        cv = jnp.where(hit, NEG, cv)

    newv = jnp.concatenate(nv + [pad3neg], axis=0)
    newi = jnp.concatenate(ni + [pad3f], axis=0)
    runv_ref[...] = newv
    runi_ref[...] = newi

    @pl.when(kk == nk - 1)
    def _finish():
        v = newv[0:K, :]
        mx = jnp.max(v, axis=0, keepdims=True)
        e = jnp.exp(v - mx)
        w = e / jnp.sum(e, axis=0, keepdims=True)
        w_ref[...] = jnp.concatenate([w, pad3f], axis=0)
        i_ref[...] = newi.astype(jnp.int32)


def _topk_tc(q, keys):
    grid = (NK // TK,)
    return pl.pallas_call(
        _topk_body,
        grid=grid,
        in_specs=[
            pl.BlockSpec((TK, ND), lambda k: (k, 0)),
            pl.BlockSpec((NQ, ND), lambda k: (0, 0)),
        ],
        out_specs=[
            pl.BlockSpec((8, NQ), lambda k: (0, 0)),
            pl.BlockSpec((8, NQ), lambda k: (0, 0)),
        ],
        out_shape=[
            jax.ShapeDtypeStruct((8, NQ), jnp.float32),
            jax.ShapeDtypeStruct((8, NQ), jnp.int32),
        ],
        scratch_shapes=[
            pltpu.VMEM((8, NQ), jnp.float32),
            pltpu.VMEM((8, NQ), jnp.float32),
        ],
        compiler_params=pltpu.CompilerParams(
            dimension_semantics=("arbitrary",),
        ),
    )(keys, q)


def _gather_sc(values, idxf, w16s):
    info = plsc.get_sparse_core_info()
    nc, ns, nl = info.num_cores, info.num_subcores, info.num_lanes
    nw = nc * ns
    b = idxf.shape[0]
    b_per_w = b // nw
    ch_q = 16
    ch_r = ch_q * K
    n_ch = b_per_w // ch_r
    dsub = ND // nl

    mesh = plsc.VectorSubcoreMesh(core_axis_name="c", subcore_axis_name="s")

    @functools.partial(
        pl.kernel,
        mesh=mesh,
        out_type=jax.ShapeDtypeStruct((NQ, ND), jnp.float32),
        scratch_types=[
            pltpu.VMEM((2, ch_r), jnp.int32),
            pltpu.VMEM((2 * ch_q, 16), jnp.float32),
            pltpu.VMEM((2 * ch_r, ND), jnp.float32),
            pltpu.VMEM((ch_q, ND), jnp.float32),
            pltpu.SemaphoreType.DMA,
            pltpu.SemaphoreType.DMA,
        ],
        compiler_params=pltpu.CompilerParams(needs_layout_passes=False),
    )
    def sc_kernel(values_hbm, idx_hbm, w_hbm, out_hbm, idx_v, w_v, rows_v,
                  out_v, sem0, sem1):
        wid = lax.axis_index("s") * nc + lax.axis_index("c")
        base = wid * b_per_w
        qb0 = base // K
        lane_iota = lax.iota(jnp.int32, nl)
        sems = (sem0, sem1)

        def start(c, bf):
            rbase = pl.multiple_of(base + c * ch_r, 8)
            qbase = pl.multiple_of(qb0 + c * ch_q, 8)
            pltpu.sync_copy(idx_hbm.at[pl.ds(rbase, ch_r)], idx_v.at[bf])
            pltpu.sync_copy(
                w_hbm.at[pl.ds(qbase, ch_q)],
                w_v.at[pl.ds(bf * ch_q, ch_q)],
            )
            return pltpu.async_copy(
                values_hbm.at[idx_v.at[bf]],
                rows_v.at[pl.ds(bf * ch_r, ch_r)],
                sems[bf],
            )

        handles = {0: start(0, 0)}
        for c in range(n_ch):
            bf = c % 2
            if c + 1 < n_ch:
                handles[c + 1] = start(c + 1, (c + 1) % 2)
            handles[c].wait()

            def q_body(qi, qcarry, bf=bf):
                wrow = w_v[bf * ch_q + qi, :]
                for j in range(K):
                    r = bf * ch_r + qi * K + j
                    wj = jnp.sum(jnp.where(lane_iota == j, wrow, 0.0))
                    w16 = jnp.broadcast_to(wj, (nl,))
                    for d in range(dsub):
                        seg = rows_v[r, pl.ds(d * nl, nl)] * w16
                        if j == 0:
                            out_v[qi, pl.ds(d * nl, nl)] = seg
                        else:
                            out_v[qi, pl.ds(d * nl, nl)] += seg
                return qcarry

            lax.fori_loop(0, ch_q, q_body, 0)
            qbase = pl.multiple_of(qb0 + c * ch_q, 8)
            pltpu.sync_copy(out_v, out_hbm.at[pl.ds(qbase, ch_q)])

    return sc_kernel(values, idxf, w16s)


def kernel(q, keys, values):
    qn = _row_normalize(q)
    kn = _row_normalize(keys)
    w8, i8 = _topk_tc(qn, kn)
    idxf = i8[:K, :].T.reshape(-1)
    w16s = jnp.pad(w8[:K, :].T, ((0, 0), (0, 11)))
    out = _gather_sc(values, idxf, w16s)
    return out[:, :, None, None]
